# baseline (speedup 1.0000x reference)
import functools

import jax
import jax.numpy as jnp
from jax import lax
from jax.experimental import pallas as pl
from jax.experimental.pallas import tpu as pltpu
from jax.experimental.pallas import tpu_sc as plsc

N = 10000
E = 320000
NODE_IN = 128
EDGE_IN = 16
H = 200
HD = H // 2
HDW = 128
LAYERS = 3
TSTEPS = 2

NC = 2
NS = 16
NW = NC * NS
C = 128
NCH = 80
EW = NCH * C
E_PAD = NW * EW
NPAD = 10112
RZ = NPAD // NS

_mesh = plsc.VectorSubcoreMesh(core_axis_name="c", subcore_axis_name="s",
                               num_cores=NC, num_subcores=NS)
_f32 = jnp.float32
_sc_params = pltpu.CompilerParams(use_tc_tiling_on_sc=False)


def _edge_prep_body(dst_hbm, ea_hbm, ones_hbm, zeros_hbm, ea_out, cnt_out,
                    acc_ea, acc_cnt, dst_v, rows_v, ones_v):
    c = lax.axis_index("c")
    s = lax.axis_index("s")
    w = c * NS + s
    zr0 = s * RZ
    pltpu.sync_copy(zeros_hbm.at[pl.ds(zr0, RZ)], acc_ea.at[pl.ds(zr0, RZ)])
    pltpu.sync_copy(zeros_hbm.at[pl.ds(zr0, RZ)], acc_cnt.at[pl.ds(zr0, RZ)])
    pltpu.sync_copy(ones_hbm, ones_v)
    pltpu.sync_copy(dst_hbm.at[pl.ds(w * NCH, NCH)], dst_v)
    plsc.subcore_barrier()

    def body(k, carry):
        off = w * EW + k * C
        pltpu.sync_copy(ea_hbm.at[pl.ds(off, C)], rows_v)
        pltpu.sync_copy(rows_v, acc_ea.at[dst_v.at[k]], add=True)
        pltpu.sync_copy(ones_v, acc_cnt.at[dst_v.at[k]], add=True)
        return carry

    lax.fori_loop(0, NCH, body, 0)
    plsc.subcore_barrier()
    pltpu.sync_copy(acc_ea.at[pl.ds(zr0, RZ)],
                    ea_out.at[pl.ds(c * NPAD + zr0, RZ)])
    pltpu.sync_copy(acc_cnt.at[pl.ds(zr0, RZ)],
                    cnt_out.at[pl.ds(c * NPAD + zr0, RZ)])


def _make_edge_prep(interpret=False):
    return pl.kernel(
        _edge_prep_body,
        out_type=[
            jax.ShapeDtypeStruct((2 * NPAD, EDGE_IN), _f32),
            jax.ShapeDtypeStruct((2 * NPAD, EDGE_IN), _f32),
        ],
        mesh=_mesh,
        scratch_types=[
            pltpu.VMEM_SHARED((NPAD, EDGE_IN), _f32),
            pltpu.VMEM_SHARED((NPAD, EDGE_IN), _f32),
            pltpu.VMEM((NCH, C), jnp.int32),
            pltpu.VMEM((C, EDGE_IN), _f32),
            pltpu.VMEM((C, EDGE_IN), _f32),
        ],
        compiler_params=_sc_params,
        interpret=interpret,
    )


_edge_prep = _make_edge_prep()


def _hsum_body(src_hbm, dst_hbm, ha_hbm, hb_hbm, zeros_hbm, out_a, out_b,
               acc, src_v, dst_v, rows_v, sem):
    c = lax.axis_index("c")
    s = lax.axis_index("s")
    w = c * NS + s
    zr0 = s * RZ
    pltpu.sync_copy(src_hbm.at[pl.ds(w * NCH, NCH)], src_v)
    pltpu.sync_copy(dst_hbm.at[pl.ds(w * NCH, NCH)], dst_v)
    for h_hbm, out in ((ha_hbm, out_a), (hb_hbm, out_b)):
        pltpu.sync_copy(zeros_hbm.at[pl.ds(zr0, RZ)], acc.at[pl.ds(zr0, RZ)])
        plsc.subcore_barrier()

        def body(k, carry):
            pltpu.async_copy(h_hbm.at[src_v.at[k]], rows_v, sem).wait()
            pltpu.sync_copy(rows_v, acc.at[dst_v.at[k]], add=True)
            return carry

        lax.fori_loop(0, NCH, body, 0)
        plsc.subcore_barrier()
        pltpu.sync_copy(acc.at[pl.ds(zr0, RZ)],
                        out.at[pl.ds(c * NPAD + zr0, RZ)])
        plsc.subcore_barrier()


def _make_hsum(interpret=False):
    return pl.kernel(
        _hsum_body,
        out_type=[
            jax.ShapeDtypeStruct((2 * NPAD, HDW), _f32),
            jax.ShapeDtypeStruct((2 * NPAD, HDW), _f32),
        ],
        mesh=_mesh,
        scratch_types=[
            pltpu.VMEM_SHARED((NPAD, HDW), _f32),
            pltpu.VMEM((NCH, C), jnp.int32),
            pltpu.VMEM((NCH, C), jnp.int32),
            pltpu.VMEM((C, HDW), _f32),
            pltpu.SemaphoreType.DMA,
        ],
        compiler_params=_sc_params,
        interpret=interpret,
    )


_hsum = _make_hsum()


BLK = 1000
GRID = N // BLK


def _prep_body(x_ref, wa_ref, ba_ref, wb_ref, bb_ref, eap_ref, cntp_ref,
               ha_ref, hb_ref, addin_ref, invdeg_ref):
    h0 = jnp.dot(x_ref[...], wa_ref[...], preferred_element_type=_f32)
    h0 = h0 + ba_ref[...]
    zpad = jnp.zeros((h0.shape[0], HDW - HD), _f32)
    ha_ref[...] = jnp.concatenate([h0[:, :HD], zpad], axis=1)
    hb_ref[...] = jnp.concatenate([h0[:, HD:], zpad], axis=1)
    ea = eap_ref[0] + eap_ref[1]
    cnt = cntp_ref[0][:, 0:1] + cntp_ref[1][:, 0:1]
    w16 = wb_ref[...].astype(jnp.bfloat16)
    ea_hi = ea.astype(jnp.bfloat16)
    ea_lo = (ea - ea_hi.astype(_f32)).astype(jnp.bfloat16)
    addin = (jnp.dot(ea_hi, w16, preferred_element_type=_f32)
             + jnp.dot(ea_lo, w16, preferred_element_type=_f32))
    addin_ref[...] = addin + cnt * bb_ref[...]
    invdeg_ref[...] = 1.0 / jnp.maximum(cnt, 1.0)


def _prep_tc(x, wa, ba, wb, bb, eap, cntp):
    return pl.pallas_call(
        _prep_body,
        grid=(GRID,),
        in_specs=[
            pl.BlockSpec((BLK, NODE_IN), lambda i: (i, 0)),
            pl.BlockSpec((NODE_IN, H), lambda i: (0, 0)),
            pl.BlockSpec((1, H), lambda i: (0, 0)),
            pl.BlockSpec((EDGE_IN, H), lambda i: (0, 0)),
            pl.BlockSpec((1, H), lambda i: (0, 0)),
            pl.BlockSpec((2, BLK, EDGE_IN), lambda i: (0, i, 0)),
            pl.BlockSpec((2, BLK, EDGE_IN), lambda i: (0, i, 0)),
        ],
        out_specs=[
            pl.BlockSpec((BLK, HDW), lambda i: (i, 0)),
            pl.BlockSpec((BLK, HDW), lambda i: (i, 0)),
            pl.BlockSpec((BLK, H), lambda i: (i, 0)),
            pl.BlockSpec((BLK, 1), lambda i: (i, 0)),
        ],
        out_shape=[
            jax.ShapeDtypeStruct((N, HDW), _f32),
            jax.ShapeDtypeStruct((N, HDW), _f32),
            jax.ShapeDtypeStruct((N, H), _f32),
            jax.ShapeDtypeStruct((N, 1), _f32),
        ],
    )(x, wa, ba, wb, bb, eap, cntp)


def _layer_body(pa_ref, pb_ref, addin_ref, invdeg_ref, ha_ref, hb_ref,
                wl_ref, bl_ref, oa_ref, ob_ref):
    seg = jnp.concatenate(
        [(pa_ref[0] + pa_ref[1])[:, :HD], (pb_ref[0] + pb_ref[1])[:, :HD]],
        axis=1)
    agg = (seg + addin_ref[...]) * invdeg_ref[...]
    z = jnp.dot(agg, wl_ref[...], preferred_element_type=_f32) + bl_ref[...]
    h = jnp.concatenate([ha_ref[..., :HD], hb_ref[..., :HD]], axis=1)
    hn = h + jnp.maximum(z, 0.0)
    zpad = jnp.zeros((hn.shape[0], HDW - HD), _f32)
    oa_ref[...] = jnp.concatenate([hn[:, :HD], zpad], axis=1)
    ob_ref[...] = jnp.concatenate([hn[:, HD:], zpad], axis=1)


def _layer_tc(pa, pb, addin, invdeg, ha, hb, wl, bl):
    return pl.pallas_call(
        _layer_body,
        grid=(GRID,),
        in_specs=[
            pl.BlockSpec((2, BLK, HDW), lambda i: (0, i, 0)),
            pl.BlockSpec((2, BLK, HDW), lambda i: (0, i, 0)),
            pl.BlockSpec((BLK, H), lambda i: (i, 0)),
            pl.BlockSpec((BLK, 1), lambda i: (i, 0)),
            pl.BlockSpec((BLK, HDW), lambda i: (i, 0)),
            pl.BlockSpec((BLK, HDW), lambda i: (i, 0)),
            pl.BlockSpec((H, H), lambda i: (0, 0)),
            pl.BlockSpec((1, H), lambda i: (0, 0)),
        ],
        out_specs=[
            pl.BlockSpec((BLK, HDW), lambda i: (i, 0)),
            pl.BlockSpec((BLK, HDW), lambda i: (i, 0)),
        ],
        out_shape=[
            jax.ShapeDtypeStruct((N, HDW), _f32),
            jax.ShapeDtypeStruct((N, HDW), _f32),
        ],
    )(pa, pb, addin, invdeg, ha, hb, wl, bl)


def _readout_body(ha_ref, hb_ref, watt_ref, batt_ref, wp_ref, bp_ref,
                  wih_ref, whh_ref, bih_ref, bhh_ref, w1_ref, b1_ref,
                  w2_ref, b2_ref, out_ref, g_ref):
    h = jnp.concatenate([ha_ref[..., :HD], hb_ref[..., :HD]], axis=1)
    g = jnp.sum(h, axis=0, keepdims=True)
    wa_c = watt_ref[0:H, :]
    wa_h = watt_ref[H:2 * H, :]
    zh = jnp.dot(h, wa_h, preferred_element_type=_f32)
    hv = jnp.dot(h, wp_ref[...], preferred_element_type=_f32) + bp_ref[...]
    for _ in range(TSTEPS):
        ctx = jnp.dot(jnp.maximum(g, 0.0), wa_c,
                      preferred_element_type=_f32)
        z = zh + batt_ref[...] + ctx
        z = jnp.where(z > 0, z, 0.01 * z)
        z = z - jnp.max(z)
        ez = jnp.exp(z)
        a = ez / jnp.sum(ez)
        g_repr = jnp.sum(a * hv, axis=0, keepdims=True)
        context = jnp.where(g_repr > 0, g_repr, jnp.exp(g_repr) - 1.0)
        gi = jnp.dot(context, wih_ref[...],
                     preferred_element_type=_f32) + bih_ref[...]
        gh = jnp.dot(g, whh_ref[...],
                     preferred_element_type=_f32) + bhh_ref[...]
        ir, iz, inn = gi[:, 0:H], gi[:, H:2 * H], gi[:, 2 * H:3 * H]
        hr, hz, hn = gh[:, 0:H], gh[:, H:2 * H], gh[:, 2 * H:3 * H]
        r = jax.nn.sigmoid(ir + hr)
        zg = jax.nn.sigmoid(iz + hz)
        ng = jnp.tanh(inn + r * hn)
        g = jnp.maximum((1.0 - zg) * ng + zg * g, 0.0)
    g_ref[...] = g
    o1 = jnp.maximum(jnp.dot(g, w1_ref[...],
                             preferred_element_type=_f32) + b1_ref[...], 0.0)
    out_ref[...] = jnp.dot(o1, w2_ref[...],
                           preferred_element_type=_f32) + b2_ref[...]


def _readout_tc(ha, hb, watt, batt, wp, bp, wih, whh, bih, bhh, w1, b1,
                w2, b2):
    return pl.pallas_call(
        _readout_body,
        out_shape=[
            jax.ShapeDtypeStruct((1, 1), _f32),
            jax.ShapeDtypeStruct((1, H), _f32),
        ],
    )(ha, hb, watt, batt, wp, bp, wih, whh, bih, bhh, w1, b1, w2, b2)


def kernel(x, edge_index, edge_attr, ecfp, params):
    src = edge_index[0]
    dst = edge_index[1]
    pad = E_PAD - E
    srcp = jnp.concatenate([src, jnp.zeros((pad,), jnp.int32)])
    srcp = srcp.reshape(NW * NCH, C)
    dstp = jnp.concatenate([dst, jnp.full((pad,), N, jnp.int32)])
    dstp2 = dstp.reshape(NW * NCH, C)
    ea16 = edge_attr.astype(jnp.bfloat16).astype(_f32)
    eap = jnp.concatenate([ea16, jnp.zeros((pad, EDGE_IN), _f32)])
    ones_c = jnp.ones((C, EDGE_IN), _f32)
    zeros_e = jnp.zeros((NPAD, EDGE_IN), _f32)
    zeros_h = jnp.zeros((NPAD, HDW), _f32)

    ea_part, cnt_part = _edge_prep(dstp2, eap, ones_c, zeros_e)
    ea_part = ea_part.reshape(2, NPAD, EDGE_IN)
    cnt_part = cnt_part.reshape(2, NPAD, EDGE_IN)

    w_atom, b_atom = params['atom']
    w_bond, b_bond = params['bond']
    ha, hb, addin, invdeg = _prep_tc(
        x, w_atom, b_atom.reshape(1, H), w_bond, b_bond.reshape(1, H),
        ea_part, cnt_part)

    for (wl, bl) in params['gcn']:
        pa, pb = _hsum(srcp, dstp2, ha, hb, zeros_h)
        pa = pa.reshape(2, NPAD, HDW)
        pb = pb.reshape(2, NPAD, HDW)
        ha, hb = _layer_tc(pa, pb, addin, invdeg, ha, hb, wl,
                           bl.reshape(1, H))

    watt, batt = params['att']
    wp, bp = params['proj']
    w1, b1 = params['out1']
    w2, b2 = params['out2']
    out, g = _readout_tc(
        ha, hb, watt, batt.reshape(1, 1), wp, bp.reshape(1, H),
        params['gru_Wih'], params['gru_Whh'],
        params['gru_bih'].reshape(1, 3 * H), params['gru_bhh'].reshape(1, 3 * H),
        w1, b1.reshape(1, 1024), w2, b2.reshape(1, 1))
    return (out, g, ecfp)


# speedup vs baseline: 1.2926x; 1.2926x over previous
import functools

import jax
import jax.numpy as jnp
from jax import lax
from jax.experimental import pallas as pl
from jax.experimental.pallas import tpu as pltpu
from jax.experimental.pallas import tpu_sc as plsc

N = 10000
E = 320000
NODE_IN = 128
EDGE_IN = 16
H = 200
HD = H // 2
HDW = 128
LAYERS = 3
TSTEPS = 2

NC = 2
NS = 16
NW = NC * NS
C = 128
NCH = 79
EW = NCH * C
E_PAD = NW * EW
NPAD = 10112
RZ = NPAD // NS

_mesh = plsc.VectorSubcoreMesh(core_axis_name="c", subcore_axis_name="s",
                               num_cores=NC, num_subcores=NS)
_f32 = jnp.float32
_sc_params = pltpu.CompilerParams(use_tc_tiling_on_sc=False)


def _edge_prep_body(dst_hbm, ea_hbm, ones_hbm, zeros_hbm, ea_out, cnt_out,
                    acc_ea, acc_cnt, dst_v, rows_v, ones_v):
    c = lax.axis_index("c")
    s = lax.axis_index("s")
    w = c * NS + s
    zr0 = s * RZ
    pltpu.sync_copy(zeros_hbm.at[pl.ds(zr0, RZ)], acc_ea.at[pl.ds(zr0, RZ)])
    pltpu.sync_copy(zeros_hbm.at[pl.ds(zr0, RZ)], acc_cnt.at[pl.ds(zr0, RZ)])
    pltpu.sync_copy(ones_hbm, ones_v)
    plsc.subcore_barrier()

    def body(k, carry):
        off = w * EW + k * C
        pltpu.sync_copy(dst_hbm.at[pl.ds(off, C)], dst_v)
        pltpu.sync_copy(ea_hbm.at[pl.ds(off, C)], rows_v)
        pltpu.sync_copy(rows_v, acc_ea.at[dst_v], add=True)
        pltpu.sync_copy(ones_v, acc_cnt.at[dst_v], add=True)
        return carry

    lax.fori_loop(0, NCH, body, 0)
    plsc.subcore_barrier()
    pltpu.sync_copy(acc_ea.at[pl.ds(zr0, RZ)],
                    ea_out.at[pl.ds(c * NPAD + zr0, RZ)])
    pltpu.sync_copy(acc_cnt.at[pl.ds(zr0, RZ)],
                    cnt_out.at[pl.ds(c * NPAD + zr0, RZ)])


def _make_edge_prep(interpret=False):
    return pl.kernel(
        _edge_prep_body,
        out_type=[
            jax.ShapeDtypeStruct((2 * NPAD, EDGE_IN), _f32),
            jax.ShapeDtypeStruct((2 * NPAD, EDGE_IN), _f32),
        ],
        mesh=_mesh,
        scratch_types=[
            pltpu.VMEM_SHARED((NPAD, EDGE_IN), _f32),
            pltpu.VMEM_SHARED((NPAD, EDGE_IN), _f32),
            pltpu.VMEM((C,), jnp.int32),
            pltpu.VMEM((C, EDGE_IN), _f32),
            pltpu.VMEM((C, EDGE_IN), _f32),
        ],
        compiler_params=_sc_params,
        interpret=interpret,
    )


_edge_prep = _make_edge_prep()


def _hsum_body(src_hbm, dst_hbm, ha_hbm, hb_hbm, zeros_hbm, out_a, out_b,
               acc, src_v, dst_v, rows_v, sem):
    c = lax.axis_index("c")
    s = lax.axis_index("s")
    w = c * NS + s
    zr0 = s * RZ
    for h_hbm, out in ((ha_hbm, out_a), (hb_hbm, out_b)):
        pltpu.sync_copy(zeros_hbm.at[pl.ds(zr0, RZ)], acc.at[pl.ds(zr0, RZ)])
        plsc.subcore_barrier()

        def body(k, carry):
            off = w * EW + k * C
            pltpu.sync_copy(src_hbm.at[pl.ds(off, C)], src_v)
            pltpu.sync_copy(dst_hbm.at[pl.ds(off, C)], dst_v)
            pltpu.async_copy(h_hbm.at[src_v], rows_v, sem).wait()
            pltpu.sync_copy(rows_v, acc.at[dst_v], add=True)
            return carry

        lax.fori_loop(0, NCH, body, 0)
        plsc.subcore_barrier()
        pltpu.sync_copy(acc.at[pl.ds(zr0, RZ)],
                        out.at[pl.ds(c * NPAD + zr0, RZ)])
        plsc.subcore_barrier()


def _make_hsum(interpret=False):
    return pl.kernel(
        _hsum_body,
        out_type=[
            jax.ShapeDtypeStruct((2 * NPAD, HDW), _f32),
            jax.ShapeDtypeStruct((2 * NPAD, HDW), _f32),
        ],
        mesh=_mesh,
        scratch_types=[
            pltpu.VMEM_SHARED((NPAD, HDW), _f32),
            pltpu.VMEM((C,), jnp.int32),
            pltpu.VMEM((C,), jnp.int32),
            pltpu.VMEM((C, HDW), _f32),
            pltpu.SemaphoreType.DMA,
        ],
        compiler_params=_sc_params,
        interpret=interpret,
    )


_hsum = _make_hsum()


BLK = 1000
GRID = N // BLK


def _prep_body(x_ref, wa_ref, ba_ref, wb_ref, bb_ref, eap_ref, cntp_ref,
               ha_ref, hb_ref, addin_ref, invdeg_ref):
    h0 = jnp.dot(x_ref[...], wa_ref[...], preferred_element_type=_f32)
    h0 = h0 + ba_ref[...]
    zpad = jnp.zeros((h0.shape[0], HDW - HD), _f32)
    ha_ref[...] = jnp.concatenate([h0[:, :HD], zpad], axis=1)
    hb_ref[...] = jnp.concatenate([h0[:, HD:], zpad], axis=1)
    ea = eap_ref[0] + eap_ref[1]
    cnt = cntp_ref[0][:, 0:1] + cntp_ref[1][:, 0:1]
    w16 = wb_ref[...].astype(jnp.bfloat16)
    ea_hi = ea.astype(jnp.bfloat16)
    ea_lo = (ea - ea_hi.astype(_f32)).astype(jnp.bfloat16)
    addin = (jnp.dot(ea_hi, w16, preferred_element_type=_f32)
             + jnp.dot(ea_lo, w16, preferred_element_type=_f32))
    addin_ref[...] = addin + cnt * bb_ref[...]
    invdeg_ref[...] = 1.0 / jnp.maximum(cnt, 1.0)


def _prep_tc(x, wa, ba, wb, bb, eap, cntp):
    return pl.pallas_call(
        _prep_body,
        grid=(GRID,),
        in_specs=[
            pl.BlockSpec((BLK, NODE_IN), lambda i: (i, 0)),
            pl.BlockSpec((NODE_IN, H), lambda i: (0, 0)),
            pl.BlockSpec((1, H), lambda i: (0, 0)),
            pl.BlockSpec((EDGE_IN, H), lambda i: (0, 0)),
            pl.BlockSpec((1, H), lambda i: (0, 0)),
            pl.BlockSpec((2, BLK, EDGE_IN), lambda i: (0, i, 0)),
            pl.BlockSpec((2, BLK, EDGE_IN), lambda i: (0, i, 0)),
        ],
        out_specs=[
            pl.BlockSpec((BLK, HDW), lambda i: (i, 0)),
            pl.BlockSpec((BLK, HDW), lambda i: (i, 0)),
            pl.BlockSpec((BLK, H), lambda i: (i, 0)),
            pl.BlockSpec((BLK, 1), lambda i: (i, 0)),
        ],
        out_shape=[
            jax.ShapeDtypeStruct((N, HDW), _f32),
            jax.ShapeDtypeStruct((N, HDW), _f32),
            jax.ShapeDtypeStruct((N, H), _f32),
            jax.ShapeDtypeStruct((N, 1), _f32),
        ],
    )(x, wa, ba, wb, bb, eap, cntp)


def _layer_body(pa_ref, pb_ref, addin_ref, invdeg_ref, ha_ref, hb_ref,
                wl_ref, bl_ref, oa_ref, ob_ref):
    seg = jnp.concatenate(
        [(pa_ref[0] + pa_ref[1])[:, :HD], (pb_ref[0] + pb_ref[1])[:, :HD]],
        axis=1)
    agg = (seg + addin_ref[...]) * invdeg_ref[...]
    z = jnp.dot(agg, wl_ref[...], preferred_element_type=_f32) + bl_ref[...]
    h = jnp.concatenate([ha_ref[..., :HD], hb_ref[..., :HD]], axis=1)
    hn = h + jnp.maximum(z, 0.0)
    zpad = jnp.zeros((hn.shape[0], HDW - HD), _f32)
    oa_ref[...] = jnp.concatenate([hn[:, :HD], zpad], axis=1)
    ob_ref[...] = jnp.concatenate([hn[:, HD:], zpad], axis=1)


def _layer_tc(pa, pb, addin, invdeg, ha, hb, wl, bl):
    return pl.pallas_call(
        _layer_body,
        grid=(GRID,),
        in_specs=[
            pl.BlockSpec((2, BLK, HDW), lambda i: (0, i, 0)),
            pl.BlockSpec((2, BLK, HDW), lambda i: (0, i, 0)),
            pl.BlockSpec((BLK, H), lambda i: (i, 0)),
            pl.BlockSpec((BLK, 1), lambda i: (i, 0)),
            pl.BlockSpec((BLK, HDW), lambda i: (i, 0)),
            pl.BlockSpec((BLK, HDW), lambda i: (i, 0)),
            pl.BlockSpec((H, H), lambda i: (0, 0)),
            pl.BlockSpec((1, H), lambda i: (0, 0)),
        ],
        out_specs=[
            pl.BlockSpec((BLK, HDW), lambda i: (i, 0)),
            pl.BlockSpec((BLK, HDW), lambda i: (i, 0)),
        ],
        out_shape=[
            jax.ShapeDtypeStruct((N, HDW), _f32),
            jax.ShapeDtypeStruct((N, HDW), _f32),
        ],
    )(pa, pb, addin, invdeg, ha, hb, wl, bl)


def _readout_body(ha_ref, hb_ref, watt_ref, batt_ref, wp_ref, bp_ref,
                  wih_ref, whh_ref, bih_ref, bhh_ref, w1_ref, b1_ref,
                  w2_ref, b2_ref, out_ref, g_ref):
    h = jnp.concatenate([ha_ref[..., :HD], hb_ref[..., :HD]], axis=1)
    g = jnp.sum(h, axis=0, keepdims=True)
    wa_c = watt_ref[0:H, :]
    wa_h = watt_ref[H:2 * H, :]
    zh = jnp.dot(h, wa_h, preferred_element_type=_f32)
    hv = jnp.dot(h, wp_ref[...], preferred_element_type=_f32) + bp_ref[...]
    for _ in range(TSTEPS):
        ctx = jnp.dot(jnp.maximum(g, 0.0), wa_c,
                      preferred_element_type=_f32)
        z = zh + batt_ref[...] + ctx
        z = jnp.where(z > 0, z, 0.01 * z)
        z = z - jnp.max(z)
        ez = jnp.exp(z)
        a = ez / jnp.sum(ez)
        g_repr = jnp.sum(a * hv, axis=0, keepdims=True)
        context = jnp.where(g_repr > 0, g_repr, jnp.exp(g_repr) - 1.0)
        gi = jnp.dot(context, wih_ref[...],
                     preferred_element_type=_f32) + bih_ref[...]
        gh = jnp.dot(g, whh_ref[...],
                     preferred_element_type=_f32) + bhh_ref[...]
        ir, iz, inn = gi[:, 0:H], gi[:, H:2 * H], gi[:, 2 * H:3 * H]
        hr, hz, hn = gh[:, 0:H], gh[:, H:2 * H], gh[:, 2 * H:3 * H]
        r = jax.nn.sigmoid(ir + hr)
        zg = jax.nn.sigmoid(iz + hz)
        ng = jnp.tanh(inn + r * hn)
        g = jnp.maximum((1.0 - zg) * ng + zg * g, 0.0)
    g_ref[...] = g
    o1 = jnp.maximum(jnp.dot(g, w1_ref[...],
                             preferred_element_type=_f32) + b1_ref[...], 0.0)
    out_ref[...] = jnp.dot(o1, w2_ref[...],
                           preferred_element_type=_f32) + b2_ref[...]


def _readout_tc(ha, hb, watt, batt, wp, bp, wih, whh, bih, bhh, w1, b1,
                w2, b2):
    return pl.pallas_call(
        _readout_body,
        out_shape=[
            jax.ShapeDtypeStruct((1, 1), _f32),
            jax.ShapeDtypeStruct((1, H), _f32),
        ],
    )(ha, hb, watt, batt, wp, bp, wih, whh, bih, bhh, w1, b1, w2, b2)


def kernel(x, edge_index, edge_attr, ecfp, params):
    src = edge_index[0]
    dst = edge_index[1]
    pad = E_PAD - E
    srcp = jnp.concatenate([src, jnp.zeros((pad,), jnp.int32)])
    dstp = jnp.concatenate([dst, jnp.full((pad,), N, jnp.int32)])
    ea16 = edge_attr.astype(jnp.bfloat16).astype(_f32)
    eap = jnp.concatenate([ea16, jnp.zeros((pad, EDGE_IN), _f32)])
    ones_c = jnp.ones((C, EDGE_IN), _f32)
    zeros_e = jnp.zeros((NPAD, EDGE_IN), _f32)
    zeros_h = jnp.zeros((NPAD, HDW), _f32)

    ea_part, cnt_part = _edge_prep(dstp, eap, ones_c, zeros_e)
    ea_part = ea_part.reshape(2, NPAD, EDGE_IN)
    cnt_part = cnt_part.reshape(2, NPAD, EDGE_IN)

    w_atom, b_atom = params['atom']
    w_bond, b_bond = params['bond']
    ha, hb, addin, invdeg = _prep_tc(
        x, w_atom, b_atom.reshape(1, H), w_bond, b_bond.reshape(1, H),
        ea_part, cnt_part)

    for (wl, bl) in params['gcn']:
        pa, pb = _hsum(srcp, dstp, ha, hb, zeros_h)
        pa = pa.reshape(2, NPAD, HDW)
        pb = pb.reshape(2, NPAD, HDW)
        ha, hb = _layer_tc(pa, pb, addin, invdeg, ha, hb, wl,
                           bl.reshape(1, H))

    watt, batt = params['att']
    wp, bp = params['proj']
    w1, b1 = params['out1']
    w2, b2 = params['out2']
    out, g = _readout_tc(
        ha, hb, watt, batt.reshape(1, 1), wp, bp.reshape(1, H),
        params['gru_Wih'], params['gru_Whh'],
        params['gru_bih'].reshape(1, 3 * H), params['gru_bhh'].reshape(1, 3 * H),
        w1, b1.reshape(1, 1024), w2, b2.reshape(1, 1))
    return (out, g, ecfp)


# speedup vs baseline: 1.2941x; 1.0012x over previous
import functools

import jax
import jax.numpy as jnp
from jax import lax
from jax.experimental import pallas as pl
from jax.experimental.pallas import tpu as pltpu
from jax.experimental.pallas import tpu_sc as plsc

N = 10000
E = 320000
NODE_IN = 128
EDGE_IN = 16
H = 200
HD = H // 2
HDW = 128
LAYERS = 3
TSTEPS = 2

NC = 2
NS = 16
NW = NC * NS
C = 128
NCH = 79
EW = NCH * C
E_PAD = NW * EW
NPAD = 10112
RZ = NPAD // NS

_mesh = plsc.VectorSubcoreMesh(core_axis_name="c", subcore_axis_name="s",
                               num_cores=NC, num_subcores=NS)
_f32 = jnp.float32
_sc_params = pltpu.CompilerParams(use_tc_tiling_on_sc=False)


def _edge_prep_body(dst_hbm, ea_hbm, ones_hbm, zeros_hbm, ea_out, cnt_out,
                    acc_ea, acc_cnt, dst_v, rows_v, ones_v):
    c = lax.axis_index("c")
    s = lax.axis_index("s")
    w = c * NS + s
    zr0 = s * RZ
    pltpu.sync_copy(zeros_hbm.at[pl.ds(zr0, RZ)], acc_ea.at[pl.ds(zr0, RZ)])
    pltpu.sync_copy(zeros_hbm.at[pl.ds(zr0, RZ)], acc_cnt.at[pl.ds(zr0, RZ)])
    pltpu.sync_copy(ones_hbm, ones_v)
    plsc.subcore_barrier()

    def body(k, carry):
        off = w * EW + k * C
        pltpu.sync_copy(dst_hbm.at[pl.ds(off, C)], dst_v)
        pltpu.sync_copy(ea_hbm.at[pl.ds(off, C)], rows_v)
        pltpu.sync_copy(rows_v, acc_ea.at[dst_v], add=True)
        pltpu.sync_copy(ones_v, acc_cnt.at[dst_v], add=True)
        return carry

    lax.fori_loop(0, NCH, body, 0)
    plsc.subcore_barrier()
    pltpu.sync_copy(acc_ea.at[pl.ds(zr0, RZ)],
                    ea_out.at[pl.ds(c * NPAD + zr0, RZ)])
    pltpu.sync_copy(acc_cnt.at[pl.ds(zr0, RZ)],
                    cnt_out.at[pl.ds(c * NPAD + zr0, RZ)])


def _make_edge_prep(interpret=False):
    return pl.kernel(
        _edge_prep_body,
        out_type=[
            jax.ShapeDtypeStruct((2 * NPAD, EDGE_IN), _f32),
            jax.ShapeDtypeStruct((2 * NPAD, EDGE_IN), _f32),
        ],
        mesh=_mesh,
        scratch_types=[
            pltpu.VMEM_SHARED((NPAD, EDGE_IN), _f32),
            pltpu.VMEM_SHARED((NPAD, EDGE_IN), _f32),
            pltpu.VMEM((C,), jnp.int32),
            pltpu.VMEM((C, EDGE_IN), _f32),
            pltpu.VMEM((C, EDGE_IN), _f32),
        ],
        compiler_params=_sc_params,
        interpret=interpret,
    )


_edge_prep = _make_edge_prep()


def _hsum_body(src_hbm, dst_hbm, ha_hbm, hb_hbm, zeros_hbm, out_a, out_b,
               acc, src_v, dst_v, rows_v, sem):
    c = lax.axis_index("c")
    s = lax.axis_index("s")
    w = c * NS + s
    zr0 = s * RZ
    for h_hbm, out in ((ha_hbm, out_a), (hb_hbm, out_b)):
        pltpu.sync_copy(zeros_hbm.at[pl.ds(zr0, RZ)], acc.at[pl.ds(zr0, RZ)])
        plsc.subcore_barrier()

        def body(k, carry):
            off = w * EW + k * C
            pltpu.sync_copy(src_hbm.at[pl.ds(off, C)], src_v)
            pltpu.sync_copy(dst_hbm.at[pl.ds(off, C)], dst_v)
            pltpu.async_copy(h_hbm.at[src_v], rows_v, sem).wait()
            pltpu.sync_copy(rows_v, acc.at[dst_v], add=True)
            return carry

        lax.fori_loop(0, NCH, body, 0)
        plsc.subcore_barrier()
        pltpu.sync_copy(acc.at[pl.ds(zr0, RZ)],
                        out.at[pl.ds(c * NPAD + zr0, RZ)])
        plsc.subcore_barrier()


def _make_hsum(interpret=False):
    return pl.kernel(
        _hsum_body,
        out_type=[
            jax.ShapeDtypeStruct((2 * NPAD, HDW), _f32),
            jax.ShapeDtypeStruct((2 * NPAD, HDW), _f32),
        ],
        mesh=_mesh,
        scratch_types=[
            pltpu.VMEM_SHARED((NPAD, HDW), _f32),
            pltpu.VMEM((C,), jnp.int32),
            pltpu.VMEM((C,), jnp.int32),
            pltpu.VMEM((C, HDW), _f32),
            pltpu.SemaphoreType.DMA,
        ],
        compiler_params=_sc_params,
        interpret=interpret,
    )


_hsum = _make_hsum()


BLK = 1000
GRID = N // BLK


def _prep_body(x_ref, wa_ref, ba_ref, wb_ref, bb_ref, eap_ref, cntp_ref,
               ha_ref, hb_ref, addin_ref, deg_ref):
    h0 = jnp.dot(x_ref[...], wa_ref[...], preferred_element_type=_f32)
    h0 = h0 + ba_ref[...]
    zpad = jnp.zeros((h0.shape[0], HDW - HD), _f32)
    ha_ref[...] = jnp.concatenate([h0[:, :HD], zpad], axis=1)
    hb_ref[...] = jnp.concatenate([h0[:, HD:], zpad], axis=1)
    ea = eap_ref[0] + eap_ref[1]
    cnt = cntp_ref[0][:, 0:1] + cntp_ref[1][:, 0:1]
    w16 = wb_ref[...].astype(jnp.bfloat16)
    ea_hi = ea.astype(jnp.bfloat16)
    ea_lo = (ea - ea_hi.astype(_f32)).astype(jnp.bfloat16)
    addin = (jnp.dot(ea_hi, w16, preferred_element_type=_f32)
             + jnp.dot(ea_lo, w16, preferred_element_type=_f32))
    addin_ref[...] = addin + cnt * bb_ref[...]
    deg_ref[...] = jnp.maximum(cnt, 1.0)


def _prep_tc(x, wa, ba, wb, bb, eap, cntp):
    return pl.pallas_call(
        _prep_body,
        grid=(GRID,),
        in_specs=[
            pl.BlockSpec((BLK, NODE_IN), lambda i: (i, 0)),
            pl.BlockSpec((NODE_IN, H), lambda i: (0, 0)),
            pl.BlockSpec((1, H), lambda i: (0, 0)),
            pl.BlockSpec((EDGE_IN, H), lambda i: (0, 0)),
            pl.BlockSpec((1, H), lambda i: (0, 0)),
            pl.BlockSpec((2, BLK, EDGE_IN), lambda i: (0, i, 0)),
            pl.BlockSpec((2, BLK, EDGE_IN), lambda i: (0, i, 0)),
        ],
        out_specs=[
            pl.BlockSpec((BLK, HDW), lambda i: (i, 0)),
            pl.BlockSpec((BLK, HDW), lambda i: (i, 0)),
            pl.BlockSpec((BLK, H), lambda i: (i, 0)),
            pl.BlockSpec((BLK, 1), lambda i: (i, 0)),
        ],
        out_shape=[
            jax.ShapeDtypeStruct((N, HDW), _f32),
            jax.ShapeDtypeStruct((N, HDW), _f32),
            jax.ShapeDtypeStruct((N, H), _f32),
            jax.ShapeDtypeStruct((N, 1), _f32),
        ],
    )(x, wa, ba, wb, bb, eap, cntp)


def _layer_body(pa_ref, pb_ref, addin_ref, deg_ref, ha_ref, hb_ref,
                wl_ref, bl_ref, oa_ref, ob_ref):
    seg = jnp.concatenate(
        [(pa_ref[0] + pa_ref[1])[:, :HD], (pb_ref[0] + pb_ref[1])[:, :HD]],
        axis=1)
    agg = (seg + addin_ref[...]) / deg_ref[...]
    z = jnp.dot(agg, wl_ref[...], preferred_element_type=_f32) + bl_ref[...]
    h = jnp.concatenate([ha_ref[..., :HD], hb_ref[..., :HD]], axis=1)
    hn = h + jnp.maximum(z, 0.0)
    zpad = jnp.zeros((hn.shape[0], HDW - HD), _f32)
    oa_ref[...] = jnp.concatenate([hn[:, :HD], zpad], axis=1)
    ob_ref[...] = jnp.concatenate([hn[:, HD:], zpad], axis=1)


def _layer_tc(pa, pb, addin, deg, ha, hb, wl, bl):
    return pl.pallas_call(
        _layer_body,
        grid=(GRID,),
        in_specs=[
            pl.BlockSpec((2, BLK, HDW), lambda i: (0, i, 0)),
            pl.BlockSpec((2, BLK, HDW), lambda i: (0, i, 0)),
            pl.BlockSpec((BLK, H), lambda i: (i, 0)),
            pl.BlockSpec((BLK, 1), lambda i: (i, 0)),
            pl.BlockSpec((BLK, HDW), lambda i: (i, 0)),
            pl.BlockSpec((BLK, HDW), lambda i: (i, 0)),
            pl.BlockSpec((H, H), lambda i: (0, 0)),
            pl.BlockSpec((1, H), lambda i: (0, 0)),
        ],
        out_specs=[
            pl.BlockSpec((BLK, HDW), lambda i: (i, 0)),
            pl.BlockSpec((BLK, HDW), lambda i: (i, 0)),
        ],
        out_shape=[
            jax.ShapeDtypeStruct((N, HDW), _f32),
            jax.ShapeDtypeStruct((N, HDW), _f32),
        ],
    )(pa, pb, addin, deg, ha, hb, wl, bl)


def _readout_body(ha_ref, hb_ref, watt_ref, batt_ref, wp_ref, bp_ref,
                  wih_ref, whh_ref, bih_ref, bhh_ref, w1_ref, b1_ref,
                  w2_ref, b2_ref, out_ref, g_ref):
    h = jnp.concatenate([ha_ref[..., :HD], hb_ref[..., :HD]], axis=1)
    g = jnp.sum(h, axis=0, keepdims=True)
    wa_c = watt_ref[0:H, :]
    wa_h = watt_ref[H:2 * H, :]
    zh = jnp.dot(h, wa_h, preferred_element_type=_f32)
    hv = jnp.dot(h, wp_ref[...], preferred_element_type=_f32) + bp_ref[...]
    for _ in range(TSTEPS):
        ctx = jnp.dot(jnp.maximum(g, 0.0), wa_c,
                      preferred_element_type=_f32)
        z = zh + batt_ref[...] + ctx
        z = jnp.where(z > 0, z, 0.01 * z)
        z = z - jnp.max(z)
        ez = jnp.exp(z)
        a = ez / jnp.sum(ez)
        g_repr = jnp.sum(a * hv, axis=0, keepdims=True)
        context = jnp.where(g_repr > 0, g_repr, jnp.exp(g_repr) - 1.0)
        gi = jnp.dot(context, wih_ref[...],
                     preferred_element_type=_f32) + bih_ref[...]
        gh = jnp.dot(g, whh_ref[...],
                     preferred_element_type=_f32) + bhh_ref[...]
        ir, iz, inn = gi[:, 0:H], gi[:, H:2 * H], gi[:, 2 * H:3 * H]
        hr, hz, hn = gh[:, 0:H], gh[:, H:2 * H], gh[:, 2 * H:3 * H]
        r = jax.nn.sigmoid(ir + hr)
        zg = jax.nn.sigmoid(iz + hz)
        ng = jnp.tanh(inn + r * hn)
        g = jnp.maximum((1.0 - zg) * ng + zg * g, 0.0)
    g_ref[...] = g
    o1 = jnp.maximum(jnp.dot(g, w1_ref[...],
                             preferred_element_type=_f32) + b1_ref[...], 0.0)
    out_ref[...] = jnp.dot(o1, w2_ref[...],
                           preferred_element_type=_f32) + b2_ref[...]


def _readout_tc(ha, hb, watt, batt, wp, bp, wih, whh, bih, bhh, w1, b1,
                w2, b2):
    return pl.pallas_call(
        _readout_body,
        out_shape=[
            jax.ShapeDtypeStruct((1, 1), _f32),
            jax.ShapeDtypeStruct((1, H), _f32),
        ],
    )(ha, hb, watt, batt, wp, bp, wih, whh, bih, bhh, w1, b1, w2, b2)


def kernel(x, edge_index, edge_attr, ecfp, params):
    src = edge_index[0]
    dst = edge_index[1]
    pad = E_PAD - E
    srcp = jnp.concatenate([src, jnp.zeros((pad,), jnp.int32)])
    dstp = jnp.concatenate([dst, jnp.full((pad,), N, jnp.int32)])
    ea16 = edge_attr.astype(jnp.bfloat16).astype(_f32)
    eap = jnp.concatenate([ea16, jnp.zeros((pad, EDGE_IN), _f32)])
    ones_c = jnp.ones((C, EDGE_IN), _f32)
    zeros_e = jnp.zeros((NPAD, EDGE_IN), _f32)
    zeros_h = jnp.zeros((NPAD, HDW), _f32)

    ea_part, cnt_part = _edge_prep(dstp, eap, ones_c, zeros_e)
    ea_part = ea_part.reshape(2, NPAD, EDGE_IN)
    cnt_part = cnt_part.reshape(2, NPAD, EDGE_IN)

    w_atom, b_atom = params['atom']
    w_bond, b_bond = params['bond']
    ha, hb, addin, deg = _prep_tc(
        x, w_atom, b_atom.reshape(1, H), w_bond, b_bond.reshape(1, H),
        ea_part, cnt_part)

    for (wl, bl) in params['gcn']:
        pa, pb = _hsum(srcp, dstp, ha, hb, zeros_h)
        pa = pa.reshape(2, NPAD, HDW)
        pb = pb.reshape(2, NPAD, HDW)
        ha, hb = _layer_tc(pa, pb, addin, deg, ha, hb, wl,
                           bl.reshape(1, H))

    watt, batt = params['att']
    wp, bp = params['proj']
    w1, b1 = params['out1']
    w2, b2 = params['out2']
    out, g = _readout_tc(
        ha, hb, watt, batt.reshape(1, 1), wp, bp.reshape(1, H),
        params['gru_Wih'], params['gru_Whh'],
        params['gru_bih'].reshape(1, 3 * H), params['gru_bhh'].reshape(1, 3 * H),
        w1, b1.reshape(1, 1024), w2, b2.reshape(1, 1))
    return (out, g, ecfp)
